# baseline (speedup 1.0000x reference)
import jax
import jax.numpy as jnp
from jax.experimental import pallas as pl
from jax.experimental.pallas import tpu as pltpu

H = 128
W = 20
B = 256
HOR = 12
UNROLL = 9
OUTP = 128


def _cell(gi, gh, hc):
    r = jax.nn.sigmoid(gi[:, :H] + gh[:, :H])
    z = jax.nn.sigmoid(gi[:, H:2 * H] + gh[:, H:2 * H])
    n = jnp.tanh(gi[:, 2 * H:] + r * gh[:, 2 * H:])
    return (1.0 - z) * n + z * hc


def _body(x0_ref, wr1_ref, bl1_ref, wr2_ref, bl2_ref,
          wih0_ref, whh0_ref, bih0_ref, bhh0_ref,
          wih1_ref, whh1_ref, bih1_ref, bhh1_ref,
          wfc_ref, fcb_ref, out_ref, seq_ref, gi_ref):
    f32 = jnp.float32
    bf16 = jnp.bfloat16

    def ghh(hc, whh_ref, bhh_ref):
        return jnp.dot(hc.astype(bf16), whh_ref[:],
                       preferred_element_type=f32) + bhh_ref[:]

    B2 = B // 2

    def l0_cell(t, hc, off):
        return _cell(gi_ref[pl.ds(t * B + off, B2), :],
                     ghh(hc, whh0_ref, bhh0_ref), hc)

    def l1_cell(x, hc):
        gi = jnp.dot(x, wih1_ref[:], preferred_element_type=f32) + bih1_ref[:]
        return _cell(gi, ghh(hc, whh1_ref, bhh1_ref), hc)

    w12 = jnp.dot(wr1_ref[:], wr2_ref[:], preferred_element_type=f32)
    b12 = jnp.dot(bl1_ref[:], wr2_ref[:], preferred_element_type=f32) + bl2_ref[:]
    seq_ref[:] = jnp.dot(x0_ref[:], w12, preferred_element_type=f32) + b12

    h0a = jnp.zeros((B2, H), f32)
    h0b = jnp.zeros((B2, H), f32)
    h1a = jnp.zeros((B2, H), f32)
    h1b = jnp.zeros((B2, H), f32)
    out_acc = jnp.zeros((B, OUTP), f32) + fcb_ref[:]

    def gi0(x):
        return jnp.dot(x, wih0_ref[:], preferred_element_type=f32) + bih0_ref[:]

    gi_ref[:] = gi0(seq_ref[:])

    for k in range(HOR):
        h0a = l0_cell(0, h0a, 0)
        h0b = l0_cell(0, h0b, B2)

        def body(i, carry):
            hc0a, hc0b, hc1a, hc1b, h0da, h0db = carry
            for j in range(UNROLL):
                t = i * UNROLL + (1 + j)
                hn0a = l0_cell(t, hc0a, 0)
                hn0b = l0_cell(t, hc0b, B2)
                hn1a = l1_cell(h0da, hc1a)
                hn1b = l1_cell(h0db, hc1b)
                seq_ref[pl.ds((t - 1) * B, B2), :] = hn1a
                seq_ref[pl.ds((t - 1) * B + B2, B2), :] = hn1b
                hc0a, hc0b, hc1a, hc1b = hn0a, hn0b, hn1a, hn1b
                h0da, h0db = hn0a, hn0b
            return hc0a, hc0b, hc1a, hc1b, h0da, h0db

        h0a, h0b, h1a, h1b, h0da, h0db = jax.lax.fori_loop(
            0, (W - 2) // UNROLL, body, (h0a, h0b, h1a, h1b, h0a, h0b))

        hn0a = l0_cell(W - 1, h0a, 0)
        hn0b = l0_cell(W - 1, h0b, B2)
        hn1a = l1_cell(h0da, h1a)
        hn1b = l1_cell(h0db, h1b)
        h1a = l1_cell(hn0a, hn1a)
        h1b = l1_cell(hn0b, hn1b)
        h0a, h0b = hn0a, hn0b
        if k < HOR - 1:
            gi_ref[pl.ds((W - 2) * B, B2), :] = gi0(hn1a)
            gi_ref[pl.ds((W - 2) * B + B2, B2), :] = gi0(hn1b)
            gi_ref[pl.ds((W - 1) * B, B2), :] = gi0(h1a)
            gi_ref[pl.ds((W - 1) * B + B2, B2), :] = gi0(h1b)
            gi_ref[pl.ds(0, (W - 2) * B), :] = gi0(
                seq_ref[pl.ds(0, (W - 2) * B), :])

        h1full = jnp.concatenate([h1a, h1b], axis=0)
        out_acc = out_acc + jnp.dot(h1full, wfc_ref[k * H:(k + 1) * H, :],
                                    preferred_element_type=f32)
    out_ref[:] = out_acc


def kernel(features, Wl1, bl1, Wr1, Wl2, bl2, Wr2, Wih0, Whh0, bih0, bhh0,
           Wih1, Whh1, bih1, bhh1, fc_w, fc_b):
    f32 = jnp.float32
    x0 = jnp.transpose(features[:, :, 0, :], (1, 0, 2)).reshape(W * B, H)
    wfc = jnp.kron(jnp.eye(HOR, OUTP, dtype=f32), fc_w.reshape(H, 1))
    fcb = jnp.broadcast_to(fc_b.reshape(1, 1), (1, OUTP))

    out = pl.pallas_call(
        _body,
        out_shape=jax.ShapeDtypeStruct((B, OUTP), f32),
        scratch_shapes=[
            pltpu.VMEM((W * B, H), f32),
            pltpu.VMEM((W * B, 3 * H), f32),
        ],
    )(x0, Wr1, bl1.reshape(1, H), Wr2, bl2.reshape(1, H),
      Wih0.T, Whh0.T.astype(jnp.bfloat16), bih0.reshape(1, 3 * H),
      bhh0.reshape(1, 3 * H),
      Wih1.T, Whh1.T.astype(jnp.bfloat16), bih1.reshape(1, 3 * H),
      bhh1.reshape(1, 3 * H),
      wfc, fcb)
    return out[:, :HOR]
